# baseline (speedup 1.0000x reference)
import functools

import jax
import jax.numpy as jnp
from jax import lax
from jax.experimental import pallas as pl
from jax.experimental.pallas import tpu as pltpu
from jax.experimental.pallas import tpu_sc as plsc

B = 16384
EMB = 64
NC, NS, L = 2, 16, 16
NW = NC * NS
SLAB = B // NW
CHUNK = 128
NCHUNK = SLAB // CHUNK
NGATHER = 10
OUT_W = 13 * EMB

GATHER_COLS = (0, 256, 320, 384, 448, 512, 576, 640, 704, 768)
DENSE_COL = 64


def _body(idx_hbm, pv_hbm, wb_hbm,
          t0, t1, t2, t3, t4, t5, t6, t7, t8, t9,
          out_hbm,
          idx_v, pv_v, wb_v, dense_v, rows_v,
          ld_sem, gsem0, gsem1, wsem0, wsem1, dsem):
    tables = (t0, t1, t2, t3, t4, t5, t6, t7, t8, t9)
    gsems = (gsem0, gsem1)
    wsems = (wsem0, wsem1)
    wid = lax.axis_index("s") * NC + lax.axis_index("c")
    base = wid * SLAB

    idx_cp = pltpu.async_copy(idx_hbm.at[wid], idx_v, ld_sem)
    pv_cp = pltpu.async_copy(pv_hbm.at[wid], pv_v, ld_sem)
    wb_cp = pltpu.async_copy(wb_hbm, wb_v, ld_sem)
    idx_cp.wait()

    gd = {}

    def fire_gathers(f, b):
        cps = []
        for j in range(NCHUNK):
            cps.append(pltpu.async_copy(
                tables[f].at[idx_v.at[f, j]],
                rows_v.at[b, pl.ds(j * CHUNK, CHUNK)],
                gsems[b]))
        gd[b] = cps

    fire_gathers(0, 0)
    fire_gathers(1, 1)

    pv_cp.wait()
    wb_cp.wait()

    ddesc = None
    for f in range(3):
        wq = [wb_v[f, pl.ds(q * L, L)] for q in range(4)]
        bq = [wb_v[3 + f, pl.ds(q * L, L)] for q in range(4)]

        def grp_body(g, carry, f=f, wq=wq, bq=bq):
            v = pv_v[f, pl.ds(g * L, L)]
            for lane in range(L):
                s = v[lane]
                for q in range(4):
                    dense_v[g * L + lane, pl.ds(q * L, L)] = s * wq[q] + bq[q]
            return carry

        if ddesc is not None:
            ddesc.wait()
        lax.fori_loop(0, SLAB // L, grp_body, 0)
        ddesc = pltpu.async_copy(
            dense_v,
            out_hbm.at[pl.ds(base, SLAB), pl.ds(DENSE_COL + f * EMB, EMB)],
            dsem)

    wd = {}
    for f in range(NGATHER):
        b = f % 2
        for cp in gd[b]:
            cp.wait()
        wd[b] = pltpu.async_copy(
            rows_v.at[b],
            out_hbm.at[pl.ds(base, SLAB), pl.ds(GATHER_COLS[f], EMB)],
            wsems[b])
        if f + 2 < NGATHER:
            wd[b].wait()
            fire_gathers(f + 2, b)

    wd[0].wait()
    wd[1].wait()
    ddesc.wait()


@functools.partial(jax.jit, static_argnums=())
def _run(idx_all, pv, wb, *tables):
    mesh = plsc.VectorSubcoreMesh(core_axis_name="c", subcore_axis_name="s")
    kfn = pl.kernel(
        _body,
        mesh=mesh,
        compiler_params=pltpu.CompilerParams(use_tc_tiling_on_sc=False),
        out_type=jax.ShapeDtypeStruct((B, OUT_W), jnp.float32),
        scratch_types=[
            pltpu.VMEM((NGATHER, NCHUNK, CHUNK), jnp.int32),
            pltpu.VMEM((3, SLAB), jnp.float32),
            pltpu.VMEM((6, EMB), jnp.float32),
            pltpu.VMEM((SLAB, EMB), jnp.float32),
            pltpu.VMEM((2, SLAB, EMB), jnp.float32),
            pltpu.SemaphoreType.DMA,
            pltpu.SemaphoreType.DMA,
            pltpu.SemaphoreType.DMA,
            pltpu.SemaphoreType.DMA,
            pltpu.SemaphoreType.DMA,
            pltpu.SemaphoreType.DMA,
        ],
    )
    return kfn(idx_all, pv, wb, *tables)


def kernel(locale, price, len_title, len_desc, encode_brand, encode_color,
           encode_size, encode_model, encode_material, encode_author,
           encode_price, encode_len_title, encode_len_desc,
           locale_table, brand_table, color_table, size_table, model_table,
           material_table, author_table, price_bin_table,
           len_title_bin_table, len_desc_bin_table,
           W_price, b_price, W_title, b_title, W_desc, b_desc):
    idx_all = jnp.stack([
        locale, encode_brand, encode_color, encode_size, encode_model,
        encode_material, encode_author, encode_price, encode_len_title,
        encode_len_desc]).astype(jnp.int32)
    idx_all = idx_all.reshape(NGATHER, NW, NCHUNK, CHUNK).transpose(1, 0, 2, 3)
    pv = jnp.stack([price, len_title, len_desc]).astype(jnp.float32)
    pv = pv.reshape(3, NW, SLAB).transpose(1, 0, 2)
    wb = jnp.concatenate([
        W_price, W_title, W_desc,
        b_price[None, :], b_title[None, :], b_desc[None, :]],
        axis=0).astype(jnp.float32)
    tables = (locale_table, brand_table, color_table, size_table,
              model_table, material_table, author_table, price_bin_table,
              len_title_bin_table, len_desc_bin_table)
    return _run(idx_all, pv, wb, *tables)


# speedup vs baseline: 1.4608x; 1.4608x over previous
import jax
import jax.numpy as jnp
from jax import lax
from jax.experimental import pallas as pl
from jax.experimental.pallas import tpu as pltpu
from jax.experimental.pallas import tpu_sc as plsc

B = 16384
EMB = 64
NC, NS, L = 2, 16, 16
NW = NC * NS
SLAB = B // NW
OUT_W = 13 * EMB


NBIG = 3
CH = 32
NCH = SLAB // CH


def _big_body(idxg_hbm, idxs_hbm, t0, t1, t2, out_hbm,
              idxg_v, idxs_v, tiles_v, rows_v, ld_sem, gsem, wsem):
    tables = (t0, t1, t2)
    wid = lax.axis_index("s") * NC + lax.axis_index("c")
    base = wid * SLAB
    cp1 = pltpu.async_copy(
        idxg_hbm.at[pl.ds(wid * NBIG * SLAB, NBIG * SLAB)], idxg_v, ld_sem)
    cp2 = pltpu.async_copy(
        idxs_hbm.at[pl.ds(wid * NBIG * SLAB, NBIG * SLAB)], idxs_v, ld_sem)
    cp1.wait()
    cp2.wait()

    for f in range(NBIG):
        def chunk_body(c, carry, f=f):
            off = pl.multiple_of(f * SLAB + c * CH, CH)
            cps = []
            for g in range(CH // L):
                tv = idxg_v[pl.ds(off + g * L, L)]
                for lane in range(L):
                    t = g * L + lane
                    cps.append(pltpu.async_copy(
                        tables[f].at[pl.ds(tv[lane], 1)],
                        tiles_v.at[pl.ds(t, 1)], gsem))
            for cp in cps:
                cp.wait()
            for g in range(CH // L):
                sv = idxs_v[pl.ds(off + g * L, L)]
                for lane in range(L):
                    s = sv[lane]
                    t = g * L + lane
                    for q in range(EMB // L):
                        rows_v[t, pl.ds(q * L, L)] = (
                            tiles_v[t, s, pl.ds(q * L, L)])
            pltpu.async_copy(
                rows_v,
                out_hbm.at[f, pl.ds(base + c * CH, CH)], wsem).wait()
            return carry

        lax.fori_loop(0, NCH, chunk_body, 0)


@jax.jit
def _big_gather(idxg, idxs, *tables):
    mesh = plsc.VectorSubcoreMesh(core_axis_name="c", subcore_axis_name="s")
    kfn = pl.kernel(
        _big_body,
        mesh=mesh,
        compiler_params=pltpu.CompilerParams(use_tc_tiling_on_sc=True),
        out_type=jax.ShapeDtypeStruct((NBIG, B, EMB), jnp.float32),
        scratch_types=[
            pltpu.VMEM((NBIG * SLAB,), jnp.int32),
            pltpu.VMEM((NBIG * SLAB,), jnp.int32),
            pltpu.VMEM((CH, 8, EMB), jnp.float32),
            pltpu.VMEM((CH, EMB), jnp.float32),
            pltpu.SemaphoreType.DMA,
            pltpu.SemaphoreType.DMA,
            pltpu.SemaphoreType.DMA,
        ],
    )
    return kfn(idxg, idxs, *tables)


NSMALL = 7
SCHUNK = 128
NSCHUNK = SLAB // SCHUNK


def _small_body(idx_hbm, t0, t1, t2, t3, t4, t5, t6, out_hbm,
                idx_v, rows_v, ld_sem, gsem0, gsem1, wsem0, wsem1):
    tables = (t0, t1, t2, t3, t4, t5, t6)
    gsems = (gsem0, gsem1)
    wsems = (wsem0, wsem1)
    wid = lax.axis_index("s") * NC + lax.axis_index("c")
    base = wid * SLAB
    pltpu.async_copy(idx_hbm.at[wid], idx_v, ld_sem).wait()

    gd = {}

    def fire_gathers(f, b):
        cps = []
        for j in range(NSCHUNK):
            cps.append(pltpu.async_copy(
                tables[f].at[idx_v.at[f, j]],
                rows_v.at[b, pl.ds(j * SCHUNK, SCHUNK)],
                gsems[b]))
        gd[b] = cps

    fire_gathers(0, 0)
    fire_gathers(1, 1)

    wd = {}
    for f in range(NSMALL):
        b = f % 2
        for cp in gd[b]:
            cp.wait()
        wd[b] = pltpu.async_copy(
            rows_v.at[b],
            out_hbm.at[f, pl.ds(base, SLAB)],
            wsems[b])
        if f + 2 < NSMALL:
            wd[b].wait()
            fire_gathers(f + 2, b)

    wd[NSMALL % 2].wait()
    wd[(NSMALL + 1) % 2].wait()


@jax.jit
def _small_gather(idx7, *tables):
    mesh = plsc.VectorSubcoreMesh(core_axis_name="c", subcore_axis_name="s")
    kfn = pl.kernel(
        _small_body,
        mesh=mesh,
        compiler_params=pltpu.CompilerParams(use_tc_tiling_on_sc=False),
        out_type=jax.ShapeDtypeStruct((NSMALL, B, EMB), jnp.float32),
        scratch_types=[
            pltpu.VMEM((NSMALL, NSCHUNK, SCHUNK), jnp.int32),
            pltpu.VMEM((2, SLAB, EMB), jnp.float32),
            pltpu.SemaphoreType.DMA,
            pltpu.SemaphoreType.DMA,
            pltpu.SemaphoreType.DMA,
            pltpu.SemaphoreType.DMA,
            pltpu.SemaphoreType.DMA,
        ],
    )
    return kfn(idx7, *tables)


BLK = 1024
STRIPES = (('s', 0), ('d', 0), ('d', 1), ('d', 2), ('b', 0), ('s', 1),
           ('s', 2), ('b', 1), ('s', 3), ('b', 2), ('s', 4), ('s', 5),
           ('s', 6))


def _tc_body(gs_ref, gb_ref, pv_ref, wb_ref, out_ref):
    for i, (kind, j) in enumerate(STRIPES):
        lo = i * EMB
        if kind == 's':
            out_ref[:, lo:lo + EMB] = gs_ref[j]
        elif kind == 'b':
            out_ref[:, lo:lo + EMB] = gb_ref[j]
        else:
            s = pv_ref[:, j:j + 1]
            out_ref[:, lo:lo + EMB] = (
                s * wb_ref[j:j + 1, :] + wb_ref[3 + j:4 + j, :])


@jax.jit
def _tc_assemble(gs, gb, pv4, wb):
    return pl.pallas_call(
        _tc_body,
        grid=(B // BLK,),
        in_specs=[
            pl.BlockSpec((NSMALL, BLK, EMB), lambda i: (0, i, 0)),
            pl.BlockSpec((NBIG, BLK, EMB), lambda i: (0, i, 0)),
            pl.BlockSpec((BLK, 4), lambda i: (i, 0)),
            pl.BlockSpec((6, EMB), lambda i: (0, 0)),
        ],
        out_specs=pl.BlockSpec((BLK, OUT_W), lambda i: (i, 0)),
        out_shape=jax.ShapeDtypeStruct((B, OUT_W), jnp.float32),
    )(gs, gb, pv4, wb)


def kernel(locale, price, len_title, len_desc, encode_brand, encode_color,
           encode_size, encode_model, encode_material, encode_author,
           encode_price, encode_len_title, encode_len_desc,
           locale_table, brand_table, color_table, size_table, model_table,
           material_table, author_table, price_bin_table,
           len_title_bin_table, len_desc_bin_table,
           W_price, b_price, W_title, b_title, W_desc, b_desc):
    idx_big = jnp.stack([encode_brand, encode_model,
                         encode_author]).astype(jnp.int32)
    idx_big = idx_big.reshape(NBIG, NW, SLAB).transpose(1, 0, 2).reshape(-1)
    gb = _big_gather(idx_big // 8, idx_big % 8,
                     brand_table.reshape(-1, 8, EMB),
                     model_table.reshape(-1, 8, EMB),
                     author_table.reshape(-1, 8, EMB))

    idx_small = jnp.stack([locale, encode_color, encode_size,
                           encode_material, encode_price, encode_len_title,
                           encode_len_desc]).astype(jnp.int32)
    idx7 = idx_small.reshape(NSMALL, NW, NSCHUNK, SCHUNK).transpose(1, 0, 2, 3)
    gs = _small_gather(idx7, locale_table, color_table, size_table,
                       material_table, price_bin_table, len_title_bin_table,
                       len_desc_bin_table)

    pv4 = jnp.stack([price, len_title, len_desc,
                     jnp.zeros_like(price)], axis=1).astype(jnp.float32)
    wb = jnp.concatenate([
        W_price, W_title, W_desc,
        b_price[None, :], b_title[None, :], b_desc[None, :]],
        axis=0).astype(jnp.float32)
    return _tc_assemble(gs, gb, pv4, wb)
